# Initial kernel scaffold; baseline (speedup 1.0000x reference)
import jax
import jax.numpy as jnp
from jax.experimental import pallas as pl


def kernel(classifications, regressions, anchors, annotations):
    raise NotImplementedError("write your pallas kernel here")


# speedup vs baseline: 2.1849x; 2.1849x over previous
import functools

import jax
import jax.numpy as jnp
from jax import lax
from jax.experimental import pallas as pl
from jax.experimental.pallas import tpu as pltpu
from jax.experimental.pallas import tpu_sc as plsc

ALPHA = 0.25
GAMMA = 2.0
EPS = 1e-4


def _make_sc_match(B, M, A_pad):
    info = plsc.get_sparse_core_info()
    NC, NS, L = info.num_cores, info.num_subcores, info.num_lanes
    NW = NC * NS
    chunk = A_pad // NW
    G = chunk // L
    mesh = plsc.VectorSubcoreMesh(core_axis_name="c", subcore_axis_name="s")

    @functools.partial(
        pl.kernel,
        mesh=mesh,
        out_type=[
            jax.ShapeDtypeStruct((B, A_pad), jnp.int32),
            jax.ShapeDtypeStruct((B, A_pad), jnp.int32),
        ],
        scratch_types=[
            pltpu.VMEM((chunk,), jnp.float32),
            pltpu.VMEM((chunk,), jnp.float32),
            pltpu.VMEM((chunk,), jnp.float32),
            pltpu.VMEM((chunk,), jnp.float32),
            pltpu.VMEM((B * M * L,), jnp.float32),
            pltpu.VMEM((B * M * L,), jnp.float32),
            pltpu.VMEM((B * M * L,), jnp.float32),
            pltpu.VMEM((B * M * L,), jnp.float32),
            pltpu.VMEM((chunk,), jnp.int32),
            pltpu.VMEM((chunk,), jnp.int32),
        ],
    )
    def sc_match(ay1, ax1, ay2, ax2, bx1, by1, bx2, by2,
                 idx_out, st_out,
                 ay1_v, ax1_v, ay2_v, ax2_v,
                 bx1_v, by1_v, bx2_v, by2_v,
                 idx_v, st_v):
        wid = lax.axis_index("s") * NC + lax.axis_index("c")
        base = wid * chunk
        pltpu.sync_copy(ay1.at[pl.ds(base, chunk)], ay1_v)
        pltpu.sync_copy(ax1.at[pl.ds(base, chunk)], ax1_v)
        pltpu.sync_copy(ay2.at[pl.ds(base, chunk)], ay2_v)
        pltpu.sync_copy(ax2.at[pl.ds(base, chunk)], ax2_v)
        pltpu.sync_copy(bx1, bx1_v)
        pltpu.sync_copy(by1, by1_v)
        pltpu.sync_copy(bx2, bx2_v)
        pltpu.sync_copy(by2, by2_v)

        for b in range(B):
            def g_body(g, carry, b=b):
                s = g * L
                ya1 = ay1_v[pl.ds(s, L)]
                xa1 = ax1_v[pl.ds(s, L)]
                ya2 = ay2_v[pl.ds(s, L)]
                xa2 = ax2_v[pl.ds(s, L)]
                area_a = (xa2 - xa1) * (ya2 - ya1)
                best = jnp.zeros((L,), jnp.float32)
                bidx = jnp.zeros((L,), jnp.int32)
                for m in range(M):
                    mo = (b * M + m) * L
                    xb1 = bx1_v[pl.ds(mo, L)]
                    yb1 = by1_v[pl.ds(mo, L)]
                    xb2 = bx2_v[pl.ds(mo, L)]
                    yb2 = by2_v[pl.ds(mo, L)]
                    area_b = (xb2 - xb1) * (yb2 - yb1)
                    iw = jnp.maximum(jnp.minimum(xa2, xb2) - jnp.maximum(xa1, xb1), 0.0)
                    ih = jnp.maximum(jnp.minimum(ya2, yb2) - jnp.maximum(ya1, yb1), 0.0)
                    inter = iw * ih
                    iou = inter / (area_a + area_b - inter)
                    take = iou > best
                    best = jnp.where(take, iou, best)
                    bidx = jnp.where(take, jnp.full((L,), m, jnp.int32), bidx)
                pos = best >= 0.5
                neg = best < 0.4
                st = jnp.where(pos, jnp.full((L,), 1, jnp.int32),
                               jnp.where(neg, jnp.full((L,), 0, jnp.int32),
                                         jnp.full((L,), -1, jnp.int32)))
                idx_v[pl.ds(s, L)] = bidx
                st_v[pl.ds(s, L)] = st
                return carry

            lax.fori_loop(0, G, g_body, 0)
            pltpu.sync_copy(idx_v, idx_out.at[b, pl.ds(base, chunk)])
            pltpu.sync_copy(st_v, st_out.at[b, pl.ds(base, chunk)])

    return sc_match


def _tc_body(A, M, BLK, nblk,
             cls_ref, reg_ref, anc_ref, ann_ref, idx_ref, st_ref,
             cls_out, reg_out, acc):
    i = pl.program_id(0)

    @pl.when(i == 0)
    def _():
        acc[...] = jnp.zeros_like(acc)

    B = cls_ref.shape[0]
    C = cls_ref.shape[2]
    a0 = i * BLK
    valid = (a0 + lax.broadcasted_iota(jnp.int32, (1, BLK), 1)) < A

    c = jnp.clip(cls_ref[...], EPS, 1.0 - EPS)
    f0 = (1.0 - ALPHA) * c * c * (-jnp.log(1.0 - c))
    s0 = jnp.sum(f0, axis=2)

    idxb = idx_ref[...]
    stb = st_ref[...]
    oh = (idxb[..., None] ==
          lax.broadcasted_iota(jnp.int32, (1, 1, M), 2)).astype(jnp.float32)
    ann = ann_ref[...]
    assigned = lax.dot_general(oh, ann, (((2,), (1,)), ((0,), (0,))),
                               preferred_element_type=jnp.float32)
    label = assigned[..., 4].astype(jnp.int32)
    cl = jnp.sum(jnp.where(lax.broadcasted_iota(jnp.int32, (1, 1, C), 2) ==
                           label[..., None], c, 0.0), axis=2)
    f0l = (1.0 - ALPHA) * cl * cl * (-jnp.log(1.0 - cl))
    f1l = ALPHA * (1.0 - cl) * (1.0 - cl) * (-jnp.log(cl))

    posb = stb == 1
    negb = stb == 0
    cls_anchor = jnp.where(posb, s0 - f0l + f1l, jnp.where(negb, s0, 0.0))
    cls_anchor = jnp.where(valid, cls_anchor, 0.0)
    cls_blk = jnp.sum(cls_anchor, axis=1)
    npos_blk = jnp.sum(jnp.where(valid & posb, 1.0, 0.0), axis=1)

    anc = anc_ref[...]
    aw = anc[:, 3] - anc[:, 1]
    ah = anc[:, 2] - anc[:, 0]
    acx = anc[:, 1] + 0.5 * aw
    acy = anc[:, 0] + 0.5 * ah
    gx1 = assigned[..., 0]
    gy1 = assigned[..., 1]
    gx2 = assigned[..., 2]
    gy2 = assigned[..., 3]
    gw = gx2 - gx1
    gh = gy2 - gy1
    gcx = gx1 + 0.5 * gw
    gcy = gy1 + 0.5 * gh
    gw = jnp.clip(gw, 1.0, None)
    gh = jnp.clip(gh, 1.0, None)
    tdx = (gcx - acx[None, :]) / aw[None, :]
    tdy = (gcy - acy[None, :]) / ah[None, :]
    tdw = jnp.log(gw / aw[None, :])
    tdh = jnp.log(gh / ah[None, :])
    reg = reg_ref[...]

    def smooth(d):
        d = jnp.abs(d)
        return jnp.where(d <= 1.0 / 9.0, 0.5 * 9.0 * d * d, d - 0.5 / 9.0)

    reg_anchor = (smooth(tdy - reg[..., 0]) + smooth(tdx - reg[..., 1]) +
                  smooth(tdh - reg[..., 2]) + smooth(tdw - reg[..., 3]))
    reg_anchor = jnp.where(valid & posb, reg_anchor, 0.0)
    reg_blk = jnp.sum(reg_anchor, axis=1)

    acc[0, :, :] = acc[0, :, :] + cls_blk[:, None]
    acc[1, :, :] = acc[1, :, :] + reg_blk[:, None]
    acc[2, :, :] = acc[2, :, :] + npos_blk[:, None]

    @pl.when(i == nblk - 1)
    def _():
        cls_s = acc[0, :, 0:1]
        reg_s = acc[1, :, 0:1]
        npos = acc[2, :, 0:1]
        cls_l = cls_s / jnp.clip(npos, 1.0, None)
        reg_l = reg_s / jnp.clip(npos * 4.0, 1.0, None)
        cls_out[...] = jnp.broadcast_to(jnp.sum(cls_l) / B, (1,))
        reg_out[...] = jnp.broadcast_to(jnp.sum(reg_l) / B, (1,))


def _tc_call(classifications, regressions, anchor, annotations, idx, st, A_pad):
    B, A, C = classifications.shape
    M = annotations.shape[1]
    BLK = 512
    nblk = A_pad // BLK
    body = functools.partial(_tc_body, A, M, BLK, nblk)
    return pl.pallas_call(
        body,
        grid=(nblk,),
        in_specs=[
            pl.BlockSpec((B, BLK, C), lambda i: (0, i, 0)),
            pl.BlockSpec((B, BLK, 4), lambda i: (0, i, 0)),
            pl.BlockSpec((BLK, 4), lambda i: (i, 0)),
            pl.BlockSpec((B, M, 5), lambda i: (0, 0, 0)),
            pl.BlockSpec((B, BLK), lambda i: (0, i)),
            pl.BlockSpec((B, BLK), lambda i: (0, i)),
        ],
        out_specs=[
            pl.BlockSpec((1,), lambda i: (0,)),
            pl.BlockSpec((1,), lambda i: (0,)),
        ],
        out_shape=[
            jax.ShapeDtypeStruct((1,), jnp.float32),
            jax.ShapeDtypeStruct((1,), jnp.float32),
        ],
        scratch_shapes=[pltpu.VMEM((3, B, 128), jnp.float32)],
        compiler_params=pltpu.CompilerParams(
            dimension_semantics=("arbitrary",)),
    )(classifications, regressions, anchor, annotations, idx, st)


def kernel(classifications, regressions, anchors, annotations):
    B, A, C = classifications.shape
    M = annotations.shape[1]
    A_pad = ((A + 511) // 512) * 512

    anchor = anchors[0]
    anc_pad = jnp.pad(anchor, ((0, A_pad - A), (0, 0)))
    ay1 = anc_pad[:, 0]
    ax1 = anc_pad[:, 1]
    ay2 = anc_pad[:, 2]
    ax2 = anc_pad[:, 3]
    def splat(col):
        v = annotations[:, :, col].reshape(B * M)
        return jnp.broadcast_to(v[:, None], (B * M, 16)).reshape(B * M * 16)

    bx1 = splat(0)
    by1 = splat(1)
    bx2 = splat(2)
    by2 = splat(3)

    sc_match = _make_sc_match(B, M, A_pad)
    idx, st = sc_match(ay1, ax1, ay2, ax2, bx1, by1, bx2, by2)

    cls_out, reg_out = _tc_call(classifications, regressions, anchor,
                                annotations, idx, st, A_pad)
    return (cls_out, reg_out)


# speedup vs baseline: 11.9012x; 5.4470x over previous
import functools

import jax
import jax.numpy as jnp
from jax import lax
from jax.experimental import pallas as pl
from jax.experimental.pallas import tpu as pltpu
from jax.experimental.pallas import tpu_sc as plsc

ALPHA = 0.25
GAMMA = 2.0
EPS = 1e-4


def _make_sc_match(B, M, A_pad):
    info = plsc.get_sparse_core_info()
    NC, NS, L = info.num_cores, info.num_subcores, info.num_lanes
    NW = NC * NS
    chunk = A_pad // NW
    G = chunk // L
    mesh = plsc.VectorSubcoreMesh(core_axis_name="c", subcore_axis_name="s")

    @functools.partial(
        pl.kernel,
        mesh=mesh,
        out_type=[
            jax.ShapeDtypeStruct((B, A_pad), jnp.int32),
            jax.ShapeDtypeStruct((B, A_pad), jnp.int32),
        ],
        scratch_types=[
            pltpu.VMEM((chunk,), jnp.float32),
            pltpu.VMEM((chunk,), jnp.float32),
            pltpu.VMEM((chunk,), jnp.float32),
            pltpu.VMEM((chunk,), jnp.float32),
            pltpu.VMEM((B * M * L,), jnp.float32),
            pltpu.VMEM((B * M * L,), jnp.float32),
            pltpu.VMEM((B * M * L,), jnp.float32),
            pltpu.VMEM((B * M * L,), jnp.float32),
            pltpu.VMEM((chunk,), jnp.int32),
            pltpu.VMEM((chunk,), jnp.int32),
        ],
    )
    def sc_match(ay1, ax1, ay2, ax2, bx1, by1, bx2, by2,
                 idx_out, st_out,
                 ay1_v, ax1_v, ay2_v, ax2_v,
                 bx1_v, by1_v, bx2_v, by2_v,
                 idx_v, st_v):
        wid = lax.axis_index("s") * NC + lax.axis_index("c")
        base = wid * chunk
        pltpu.sync_copy(ay1.at[pl.ds(base, chunk)], ay1_v)
        pltpu.sync_copy(ax1.at[pl.ds(base, chunk)], ax1_v)
        pltpu.sync_copy(ay2.at[pl.ds(base, chunk)], ay2_v)
        pltpu.sync_copy(ax2.at[pl.ds(base, chunk)], ax2_v)
        pltpu.sync_copy(bx1, bx1_v)
        pltpu.sync_copy(by1, by1_v)
        pltpu.sync_copy(bx2, bx2_v)
        pltpu.sync_copy(by2, by2_v)

        for b in range(B):
            def g_body(g, carry, b=b):
                s = g * L
                ya1 = ay1_v[pl.ds(s, L)]
                xa1 = ax1_v[pl.ds(s, L)]
                ya2 = ay2_v[pl.ds(s, L)]
                xa2 = ax2_v[pl.ds(s, L)]
                area_a = (xa2 - xa1) * (ya2 - ya1)
                best = jnp.zeros((L,), jnp.float32)
                bidx = jnp.zeros((L,), jnp.int32)
                for m in range(M):
                    mo = (b * M + m) * L
                    xb1 = bx1_v[pl.ds(mo, L)]
                    yb1 = by1_v[pl.ds(mo, L)]
                    xb2 = bx2_v[pl.ds(mo, L)]
                    yb2 = by2_v[pl.ds(mo, L)]
                    area_b = (xb2 - xb1) * (yb2 - yb1)
                    iw = jnp.maximum(jnp.minimum(xa2, xb2) - jnp.maximum(xa1, xb1), 0.0)
                    ih = jnp.maximum(jnp.minimum(ya2, yb2) - jnp.maximum(ya1, yb1), 0.0)
                    inter = iw * ih
                    iou = inter / (area_a + area_b - inter)
                    take = iou > best
                    best = jnp.where(take, iou, best)
                    bidx = jnp.where(take, jnp.full((L,), m, jnp.int32), bidx)
                pos = best >= 0.5
                neg = best < 0.4
                st = jnp.where(pos, jnp.full((L,), 1, jnp.int32),
                               jnp.where(neg, jnp.full((L,), 0, jnp.int32),
                                         jnp.full((L,), -1, jnp.int32)))
                idx_v[pl.ds(s, L)] = bidx
                st_v[pl.ds(s, L)] = st
                return carry

            lax.fori_loop(0, G, g_body, 0)
            pltpu.sync_copy(idx_v, idx_out.at[b, pl.ds(base, chunk)])
            pltpu.sync_copy(st_v, st_out.at[b, pl.ds(base, chunk)])

    return sc_match


def _tc_body(A, M, BLK, nblk,
             cls_ref, regt_ref, anct_ref, annt_ref, idx_ref, st_ref,
             cls_out, reg_out, acc):
    i = pl.program_id(0)

    @pl.when(i == 0)
    def _():
        acc[...] = jnp.zeros_like(acc)

    B = cls_ref.shape[0]
    C = cls_ref.shape[1]
    a0 = i * BLK
    valid = (a0 + lax.broadcasted_iota(jnp.int32, (1, BLK), 1)) < A

    c = jnp.clip(cls_ref[...], EPS, 1.0 - EPS)
    f0 = c * c * (-jnp.log(1.0 - c))
    s0 = (1.0 - ALPHA) * jnp.sum(f0, axis=1)

    idxb = idx_ref[...]
    stb = st_ref[...]
    oht = (idxb[:, None, :] ==
           lax.broadcasted_iota(jnp.int32, (1, M, 1), 1)).astype(jnp.float32)
    annt = annt_ref[...]
    assigned = lax.dot_general(annt, oht, (((2,), (1,)), ((0,), (0,))),
                               preferred_element_type=jnp.float32)
    label = assigned[:, 4, :].astype(jnp.int32)
    ohc = label[:, None, :] == lax.broadcasted_iota(jnp.int32, (1, C, 1), 1)
    cl = jnp.sum(jnp.where(ohc, c, 0.0), axis=1)
    f0l = (1.0 - ALPHA) * cl * cl * (-jnp.log(1.0 - cl))
    f1l = ALPHA * (1.0 - cl) * (1.0 - cl) * (-jnp.log(cl))

    posb = stb == 1
    keep = valid & (stb != -1)
    posv = valid & posb
    cls_anchor = (jnp.where(keep, s0, 0.0) +
                  jnp.where(posv, f1l - f0l, 0.0))

    anct = anct_ref[...]
    aw = anct[3, :] - anct[1, :]
    ah = anct[2, :] - anct[0, :]
    acx = anct[1, :] + 0.5 * aw
    acy = anct[0, :] + 0.5 * ah
    gx1 = assigned[:, 0, :]
    gy1 = assigned[:, 1, :]
    gx2 = assigned[:, 2, :]
    gy2 = assigned[:, 3, :]
    gw = gx2 - gx1
    gh = gy2 - gy1
    gcx = gx1 + 0.5 * gw
    gcy = gy1 + 0.5 * gh
    gw = jnp.clip(gw, 1.0, None)
    gh = jnp.clip(gh, 1.0, None)
    tdx = (gcx - acx[None, :]) / aw[None, :]
    tdy = (gcy - acy[None, :]) / ah[None, :]
    tdw = jnp.log(gw / aw[None, :])
    tdh = jnp.log(gh / ah[None, :])
    regt = regt_ref[...]

    def smooth(d):
        d = jnp.abs(d)
        return jnp.where(d <= 1.0 / 9.0, 0.5 * 9.0 * d * d, d - 0.5 / 9.0)

    reg_anchor = (smooth(tdy - regt[:, 0, :]) + smooth(tdx - regt[:, 1, :]) +
                  smooth(tdh - regt[:, 2, :]) + smooth(tdw - regt[:, 3, :]))
    reg_anchor = jnp.where(posv, reg_anchor, 0.0)

    acc[0, :, :] = acc[0, :, :] + cls_anchor
    acc[1, :, :] = acc[1, :, :] + reg_anchor
    acc[2, :, :] = acc[2, :, :] + jnp.where(posv, 1.0, 0.0)

    @pl.when(i == nblk - 1)
    def _():
        cls_s = jnp.sum(acc[0, :, :], axis=1)
        reg_s = jnp.sum(acc[1, :, :], axis=1)
        npos = jnp.sum(acc[2, :, :], axis=1)
        cls_l = cls_s / jnp.clip(npos, 1.0, None)
        reg_l = reg_s / jnp.clip(npos * 4.0, 1.0, None)
        cls_out[...] = jnp.broadcast_to(jnp.sum(cls_l) / B, (1,))
        reg_out[...] = jnp.broadcast_to(jnp.sum(reg_l) / B, (1,))


def _tc_call(classifications_t, regressions_t, anchor_t, annotations_t, idx,
             st, A_pad):
    B, C, A = classifications_t.shape
    M = annotations_t.shape[2]
    BLK = 512
    nblk = A_pad // BLK
    body = functools.partial(_tc_body, A, M, BLK, nblk)
    return pl.pallas_call(
        body,
        grid=(nblk,),
        in_specs=[
            pl.BlockSpec((B, C, BLK), lambda i: (0, 0, i)),
            pl.BlockSpec((B, 4, BLK), lambda i: (0, 0, i)),
            pl.BlockSpec((4, BLK), lambda i: (0, i)),
            pl.BlockSpec((B, 5, M), lambda i: (0, 0, 0)),
            pl.BlockSpec((B, BLK), lambda i: (0, i)),
            pl.BlockSpec((B, BLK), lambda i: (0, i)),
        ],
        out_specs=[
            pl.BlockSpec((1,), lambda i: (0,)),
            pl.BlockSpec((1,), lambda i: (0,)),
        ],
        out_shape=[
            jax.ShapeDtypeStruct((1,), jnp.float32),
            jax.ShapeDtypeStruct((1,), jnp.float32),
        ],
        scratch_shapes=[pltpu.VMEM((3, B, BLK), jnp.float32)],
        compiler_params=pltpu.CompilerParams(
            dimension_semantics=("arbitrary",)),
    )(classifications_t, regressions_t, anchor_t, annotations_t, idx, st)


def kernel(classifications, regressions, anchors, annotations):
    B, A, C = classifications.shape
    M = annotations.shape[1]
    A_pad = ((A + 511) // 512) * 512

    anchor = anchors[0]
    anc_pad = jnp.pad(anchor, ((0, A_pad - A), (0, 0)))
    ay1 = anc_pad[:, 0]
    ax1 = anc_pad[:, 1]
    ay2 = anc_pad[:, 2]
    ax2 = anc_pad[:, 3]
    def splat(col):
        v = annotations[:, :, col].reshape(B * M)
        return jnp.broadcast_to(v[:, None], (B * M, 16)).reshape(B * M * 16)

    bx1 = splat(0)
    by1 = splat(1)
    bx2 = splat(2)
    by2 = splat(3)

    sc_match = _make_sc_match(B, M, A_pad)
    idx, st = sc_match(ay1, ax1, ay2, ax2, bx1, by1, bx2, by2)

    classifications_t = jnp.transpose(classifications, (0, 2, 1))
    regressions_t = jnp.transpose(regressions, (0, 2, 1))
    anchor_t = jnp.transpose(anc_pad, (1, 0))
    annotations_t = jnp.transpose(annotations, (0, 2, 1))
    cls_out, reg_out = _tc_call(classifications_t, regressions_t, anchor_t,
                                annotations_t, idx, st, A_pad)
    return (cls_out, reg_out)
